# baseline (speedup 1.0000x reference)
import functools

import jax
import jax.numpy as jnp
from jax import lax
from jax.experimental import pallas as pl
from jax.experimental.pallas import tpu as pltpu, tpu_sc as plsc

VOCAB = 49408
HIDDEN = 512
MAX_POS = 77
NC = 2
NS = 16
NW = NC * NS
LANES = 16
NB = 32
NBUF = 5
PF = 3
PE_HALF = 40


def _sc_embed(ids3, table, pe):
    mesh = plsc.VectorSubcoreMesh(core_axis_name="c", subcore_axis_name="s")

    @functools.partial(
        pl.kernel,
        mesh=mesh,
        out_type=jax.ShapeDtypeStruct(
            (MAX_POS, NW * NB, HIDDEN), jnp.float32),
        scratch_types=(
            [pltpu.VMEM((MAX_POS, NB), jnp.int32),
             pltpu.VMEM((PE_HALF, HIDDEN), jnp.float32)]
            + [pltpu.VMEM((NB, HIDDEN), jnp.float32)] * NBUF
            + [pltpu.SemaphoreType.DMA] * (2 * NBUF)
        ),
    )
    def k(ids_hbm, table_hbm, pe_hbm, out_hbm, idx_v, pe_v, *bufs):
        rows = bufs[:NBUF]
        gsems = bufs[NBUF:2 * NBUF]
        ssems = bufs[2 * NBUF:]
        wid = lax.axis_index("s") * NC + lax.axis_index("c")
        base = wid * NB
        pltpu.sync_copy(ids_hbm.at[wid], idx_v)
        for s0 in range(PF):
            pltpu.async_copy(table_hbm.at[idx_v.at[s0]], rows[s0],
                             gsems[s0])
        pltpu.sync_copy(pe_hbm.at[pl.ds(0, PE_HALF)], pe_v)

        def quad_body(i, carry):
            for b in range(NBUF):
                s = NBUF * i + b

                @pl.when(s < MAX_POS)
                def _():
                    pltpu.make_async_copy(
                        table_hbm.at[idx_v.at[s]], rows[b], gsems[b]).wait()

                    b2 = (b + PF) % NBUF

                    @pl.when(s + PF < MAX_POS)
                    def _():
                        @pl.when(s >= NBUF - PF)
                        def _():
                            pltpu.make_async_copy(
                                rows[b2],
                                out_hbm.at[s - (NBUF - PF),
                                           pl.ds(base, NB)],
                                ssems[b2]).wait()

                        pltpu.async_copy(
                            table_hbm.at[idx_v.at[s + PF]], rows[b2],
                            gsems[b2])

                    @pl.when(s == PE_HALF)
                    def _():
                        pltpu.sync_copy(
                            pe_hbm.at[pl.ds(PE_HALF, PE_HALF)], pe_v)

                    p = jnp.where(s >= PE_HALF, s - PE_HALF, s)
                    pes = [pe_v[p, pl.ds(j * LANES, LANES)]
                           for j in range(HIDDEN // LANES)]

                    @plsc.parallel_loop(0, NB, step=1, unroll=4)
                    def _(r):
                        for j in range(HIDDEN // LANES):
                            plsc.addupdate(
                                rows[b].at[r, pl.ds(j * LANES, LANES)],
                                pes[j])

                    pltpu.async_copy(
                        rows[b], out_hbm.at[s, pl.ds(base, NB)], ssems[b])
            return carry

        lax.fori_loop(0, (MAX_POS + NBUF - 1) // NBUF, quad_body, 0,
                      unroll=1)

        for s in range(MAX_POS - NBUF, MAX_POS):
            pltpu.make_async_copy(
                rows[s % NBUF], out_hbm.at[s, pl.ds(base, NB)],
                ssems[s % NBUF]).wait()

    return k(ids3, table, pe)


def kernel(input_ids, token_table, pos_emb):
    batch, seq = input_ids.shape
    ids3 = (input_ids.astype(jnp.int32).T
            .reshape(seq, NW, NB).transpose(1, 0, 2))
    pe = pos_emb.astype(jnp.float32).reshape(MAX_POS, HIDDEN)
    pe = jnp.pad(pe, ((0, 2 * PE_HALF - MAX_POS), (0, 0)))
    out = _sc_embed(ids3, token_table.astype(jnp.float32), pe)
    return out.transpose(1, 0, 2)


# speedup vs baseline: 1.0184x; 1.0184x over previous
import functools

import jax
import jax.numpy as jnp
from jax import lax
from jax.experimental import pallas as pl
from jax.experimental.pallas import tpu as pltpu, tpu_sc as plsc

VOCAB = 49408
HIDDEN = 512
MAX_POS = 77
NC = 2
NS = 16
NW = NC * NS
LANES = 16
NB = 32
NBUF = 4
PF = 2


def _sc_embed(ids3, table, pe):
    mesh = plsc.VectorSubcoreMesh(core_axis_name="c", subcore_axis_name="s")

    @functools.partial(
        pl.kernel,
        mesh=mesh,
        out_type=jax.ShapeDtypeStruct(
            (MAX_POS, NW * NB, HIDDEN), jnp.float32),
        scratch_types=(
            [pltpu.VMEM((MAX_POS, NB), jnp.int32),
             pltpu.VMEM((MAX_POS, HIDDEN), jnp.float32)]
            + [pltpu.VMEM((NB, HIDDEN), jnp.float32)] * NBUF
            + [pltpu.SemaphoreType.DMA] * (2 * NBUF)
        ),
    )
    def k(ids_hbm, table_hbm, pe_hbm, out_hbm, idx_v, pe_v, *bufs):
        rows = bufs[:NBUF]
        gsems = bufs[NBUF:2 * NBUF]
        ssems = bufs[2 * NBUF:]
        wid = lax.axis_index("s") * NC + lax.axis_index("c")
        base = wid * NB
        pltpu.sync_copy(ids_hbm.at[wid], idx_v)
        for s0 in range(PF):
            pltpu.async_copy(table_hbm.at[idx_v.at[s0]], rows[s0],
                             gsems[s0])
        pltpu.sync_copy(pe_hbm, pe_v)

        def quad_body(i, carry):
            for b in range(NBUF):
                s = NBUF * i + b

                @pl.when(s < MAX_POS)
                def _():
                    pltpu.make_async_copy(
                        table_hbm.at[idx_v.at[s]], rows[b], gsems[b]).wait()

                    b2 = (b + PF) % NBUF

                    @pl.when(s + PF < MAX_POS)
                    def _():
                        @pl.when(s >= NBUF - PF)
                        def _():
                            pltpu.make_async_copy(
                                rows[b2],
                                out_hbm.at[s - (NBUF - PF),
                                           pl.ds(base, NB)],
                                ssems[b2]).wait()

                        pltpu.async_copy(
                            table_hbm.at[idx_v.at[s + PF]], rows[b2],
                            gsems[b2])

                    pes = [pe_v[s, pl.ds(j * LANES, LANES)]
                           for j in range(HIDDEN // LANES)]

                    @plsc.parallel_loop(0, NB, step=1, unroll=4)
                    def _(r):
                        for j in range(HIDDEN // LANES):
                            plsc.addupdate(
                                rows[b].at[r, pl.ds(j * LANES, LANES)],
                                pes[j])

                    pltpu.async_copy(
                        rows[b], out_hbm.at[s, pl.ds(base, NB)], ssems[b])
            return carry

        lax.fori_loop(0, (MAX_POS + NBUF - 1) // NBUF, quad_body, 0,
                      unroll=1)

        for s in range(MAX_POS - NBUF, MAX_POS):
            pltpu.make_async_copy(
                rows[s % NBUF], out_hbm.at[s, pl.ds(base, NB)],
                ssems[s % NBUF]).wait()

    return k(ids3, table, pe)


def kernel(input_ids, token_table, pos_emb):
    batch, seq = input_ids.shape
    ids3 = (input_ids.astype(jnp.int32).T
            .reshape(seq, NW, NB).transpose(1, 0, 2))
    pe = pos_emb.astype(jnp.float32).reshape(MAX_POS, HIDDEN)
    out = _sc_embed(ids3, token_table.astype(jnp.float32), pe)
    return out.transpose(1, 0, 2)
